# baseline (speedup 1.0000x reference)
import functools

import jax
import jax.numpy as jnp
from jax import lax
from jax.experimental import pallas as pl
from jax.experimental.pallas import tpu as pltpu
from jax.experimental.pallas import tpu_sc as plsc

BLANK = 0
W = 16
TOP = 4
NEG = -1.0e30
T = 256
B = 16
C = 64
L = 16


def _logaddexp(a, b):
    m = jnp.maximum(a, b)
    d = -jnp.abs(a - b)
    e = jnp.exp(d)
    w = e / (e + 2.0)
    t = w * w
    p = jnp.full_like(w, 1.0 / 15.0)
    for coef in (1.0 / 13.0, 1.0 / 11.0, 1.0 / 9.0, 1.0 / 7.0, 1.0 / 5.0,
                 1.0 / 3.0, 1.0):
        p = p * t + coef
    return m + 2.0 * w * p


def _splat(x, dtype=jnp.int32):
    return jnp.zeros((L,), dtype) + x


def _vtake(x, idx):
    return x.at[idx].get(mode="promise_in_bounds")


def _body(lp_hbm, len_hbm, out_hbm, lp_v, len_v, bp_par, bp_chr, labels_v):
    cid = lax.axis_index("c")
    sid = lax.axis_index("s")
    b = sid * 2 + cid

    @pl.when(b < B)
    def _():
        pltpu.sync_copy(lp_hbm.at[b], lp_v)
        pltpu.sync_copy(len_hbm, len_v)
        lanes = lax.iota(jnp.int32, L)
        my_len = plsc.load_gather(len_v, [_splat(b)])[0]

        neg = jnp.full((L,), NEG, jnp.float32)
        pb0 = jnp.where(lanes == 0, 0.0, NEG).astype(jnp.float32)
        last0 = jnp.full((L,), -1, jnp.int32)
        len0 = jnp.zeros((L,), jnp.int32)

        def step(t, carry):
            pb, pnb, last, lens = carry
            tot = _logaddexp(pb, pnb)
            lp0 = plsc.load_gather(lp_v, [_splat(t), _splat(0)])
            safe_last = jnp.maximum(last, 0)
            lplast = plsc.load_gather(lp_v, [_splat(t), safe_last])
            s_pb = tot + lp0
            s_pnb = jnp.where(last >= 0, pnb + lplast, NEG)
            stay = _logaddexp(s_pb, s_pnb)

            def merge_top(a, b):
                av, ae = a
                bv, be = b
                rbv = lax.rev(bv, (0,))
                rbe = lax.rev(be, (0,))
                keep = av >= rbv
                top = (jnp.where(keep, av, rbv), jnp.where(keep, ae, rbe))
                bot = (jnp.where(keep, rbv, av), jnp.where(keep, rbe, ae))
                return tuple(plsc.sort_key_val(*top, descending=True)), bot

            lpch = []
            for j in range(4):
                cls = lanes + j * L
                v = plsc.load_gather(lp_v, [_splat(t), cls])
                if j == 0:
                    v = jnp.where(lanes == 0, NEG, v)
                lpch.append(tuple(plsc.sort_key_val(v, cls, descending=True)))
            ab, ab_bot = merge_top(lpch[0], lpch[1])
            cd, cd_bot = merge_top(lpch[2], lpch[3])
            s16, fin_bot = merge_top(ab, cd)
            k1 = ab_bot[0] >= cd_bot[0]
            m1v = jnp.where(k1, ab_bot[0], cd_bot[0])
            m1i = jnp.where(k1, ab_bot[1], cd_bot[1])
            k2 = m1v >= fin_bot[0]
            s17v, s17i = plsc.sort_key_val(
                jnp.where(k2, m1v, fin_bot[0]),
                jnp.where(k2, m1i, fin_bot[1]), descending=True)

            chunks = [(stay, lanes * C)]
            for j in range(17):
                if j < 16:
                    sj = _vtake(s16[1], _splat(j))
                    lpj = _vtake(s16[0], _splat(j))
                else:
                    sj = _vtake(s17i, _splat(0))
                    lpj = _vtake(s17v, _splat(0))
                score = jnp.where(last == sj, pb, tot) + lpj
                chunks.append((score, lanes * C + sj))

            level = [tuple(plsc.sort_key_val(v, e, descending=True))
                     for v, e in chunks]
            while len(level) > 1:
                nxt = []
                for i in range(0, len(level) - 1, 2):
                    nxt.append(merge_top(level[i], level[i + 1])[0])
                if len(level) % 2:
                    nxt.append(level[-1])
                level = nxt
            cur_v, cur_e = level[0]

            sel_w = lax.shift_right_logical(cur_e, 6)
            sel_c = jnp.bitwise_and(cur_e, C - 1)
            g_pb = _vtake(pb, sel_w)
            g_spb = _vtake(s_pb, sel_w)
            g_spnb = _vtake(s_pnb, sel_w)
            g_tot = _vtake(tot, sel_w)
            g_last = _vtake(last, sel_w)
            g_len = _vtake(lens, sel_w)
            lp_sel = plsc.load_gather(lp_v, [_splat(t), sel_c])
            e_pnb = jnp.where(sel_c == g_last, g_pb, g_tot) + lp_sel
            is_stay = sel_c == 0
            new_pb = jnp.where(is_stay, g_spb, NEG).astype(jnp.float32)
            new_pnb = jnp.where(is_stay, g_spnb, e_pnb)
            new_last = jnp.where(is_stay, g_last, sel_c)
            new_len = jnp.where(is_stay, g_len, jnp.minimum(g_len + 1, T))
            bp_par[t] = sel_w
            bp_chr[t] = jnp.where(is_stay, -1, sel_c)
            return new_pb, new_pnb, new_last, new_len

        def two_steps(i, carry):
            return step(2 * i + 1, step(2 * i, carry))

        carry = lax.fori_loop(
            0, my_len // 2, two_steps, (pb0, neg, last0, len0))
        pb, pnb, last, lens = lax.fori_loop(
            2 * (my_len // 2), my_len, step, carry)

        score = _logaddexp(pb, pnb)
        sorted_sc, _ = plsc.sort_key_val(score, lanes, descending=True)
        mx = _vtake(sorted_sc, _splat(0))
        wstar = plsc.all_reduce_ffs(score == mx)
        pos0 = _vtake(lens, _splat(0) + wstar)

        for j in range(T // L):
            labels_v[pl.ds(j * L, L)] = jnp.zeros((L,), jnp.int32)

        def back(i, carry):
            wv, pos = carry
            t = my_len - 1 - i
            chr_ = plsc.load_gather(bp_chr, [_splat(t), wv])
            par = plsc.load_gather(bp_par, [_splat(t), wv])
            emit = chr_ >= 0
            npos = pos - jnp.where(emit, 1, 0)
            plsc.store_scatter(labels_v, [npos], chr_,
                               mask=jnp.logical_and(lanes == 0, emit))
            return par, npos

        lax.fori_loop(0, my_len, back, (_splat(0) + wstar, pos0))
        pltpu.sync_copy(labels_v, out_hbm.at[b])


@jax.jit
def kernel(data, data_length):
    lp = jnp.transpose(data, (1, 0, 2))
    mesh = plsc.VectorSubcoreMesh(core_axis_name="c", subcore_axis_name="s")
    f = pl.kernel(
        _body,
        out_type=jax.ShapeDtypeStruct((B, T), jnp.int32),
        mesh=mesh,
        compiler_params=pltpu.CompilerParams(needs_layout_passes=False),
        scratch_types=[
            pltpu.VMEM((T, C), jnp.float32),
            pltpu.VMEM((L,), jnp.int32),
            pltpu.VMEM((T, W), jnp.int32),
            pltpu.VMEM((T, W), jnp.int32),
            pltpu.VMEM((T,), jnp.int32),
        ],
    )
    return f(lp, data_length)


# speedup vs baseline: 1.0109x; 1.0109x over previous
import functools

import jax
import jax.numpy as jnp
from jax import lax
from jax.experimental import pallas as pl
from jax.experimental.pallas import tpu as pltpu
from jax.experimental.pallas import tpu_sc as plsc

BLANK = 0
W = 16
TOP = 4
NEG = -1.0e30
T = 256
B = 16
C = 64
L = 16


def _logaddexp(a, b):
    m = jnp.maximum(a, b)
    d = -jnp.abs(a - b)
    e = jnp.exp(d)
    w = e / (e + 2.0)
    t = w * w
    p = jnp.full_like(w, 1.0 / 15.0)
    for coef in (1.0 / 13.0, 1.0 / 11.0, 1.0 / 9.0, 1.0 / 7.0, 1.0 / 5.0,
                 1.0 / 3.0, 1.0):
        p = p * t + coef
    return m + 2.0 * w * p


def _splat(x, dtype=jnp.int32):
    return jnp.zeros((L,), dtype) + x


def _vtake(x, idx):
    return x.at[idx].get(mode="promise_in_bounds")


def _body(lp_hbm, len_hbm, out_hbm, lp_v, len_v, bp_par, bp_chr, labels_v):
    cid = lax.axis_index("c")
    sid = lax.axis_index("s")
    b = sid * 2 + cid

    @pl.when(b < B)
    def _():
        pltpu.sync_copy(lp_hbm.at[b], lp_v)
        pltpu.sync_copy(len_hbm, len_v)
        lanes = lax.iota(jnp.int32, L)
        my_len = plsc.load_gather(len_v, [_splat(b)])[0]

        neg = jnp.full((L,), NEG, jnp.float32)
        pb0 = jnp.where(lanes == 0, 0.0, NEG).astype(jnp.float32)
        last0 = jnp.full((L,), -1, jnp.int32)
        len0 = jnp.zeros((L,), jnp.int32)

        def step(t, carry):
            pb, pnb, last, lens = carry
            tot = _logaddexp(pb, pnb)
            lp0 = plsc.load_gather(lp_v, [_splat(t), _splat(0)])
            safe_last = jnp.maximum(last, 0)
            lplast = plsc.load_gather(lp_v, [_splat(t), safe_last])
            s_pb = tot + lp0
            s_pnb = jnp.where(last >= 0, pnb + lplast, NEG)
            stay = _logaddexp(s_pb, s_pnb)

            def merge_top(a, b):
                av, ae = a
                bv, be = b
                rbv = lax.rev(bv, (0,))
                rbe = lax.rev(be, (0,))
                keep = av >= rbv
                top = (jnp.where(keep, av, rbv), jnp.where(keep, ae, rbe))
                bot = (jnp.where(keep, rbv, av), jnp.where(keep, rbe, ae))
                return tuple(plsc.sort_key_val(*top, descending=True)), bot

            lpch = []
            for j in range(4):
                cls = lanes + j * L
                v = plsc.load_gather(lp_v, [_splat(t), cls])
                if j == 0:
                    v = jnp.where(lanes == 0, NEG, v)
                lpch.append(tuple(plsc.sort_key_val(v, cls, descending=True)))
            ab, ab_bot = merge_top(lpch[0], lpch[1])
            cd, cd_bot = merge_top(lpch[2], lpch[3])
            s16, fin_bot = merge_top(ab, cd)
            k1 = ab_bot[0] >= cd_bot[0]
            m1v = jnp.where(k1, ab_bot[0], cd_bot[0])
            m1i = jnp.where(k1, ab_bot[1], cd_bot[1])
            k2 = m1v >= fin_bot[0]
            s17v, s17i = plsc.sort_key_val(
                jnp.where(k2, m1v, fin_bot[0]),
                jnp.where(k2, m1i, fin_bot[1]), descending=True)

            chunks = [(stay, lanes * C)]
            for j in range(17):
                if j < 16:
                    sj = _vtake(s16[1], _splat(j))
                    lpj = _vtake(s16[0], _splat(j))
                else:
                    sj = _vtake(s17i, _splat(0))
                    lpj = _vtake(s17v, _splat(0))
                score = jnp.where(last == sj, pb, tot) + lpj
                chunks.append((score, lanes * C + sj))

            level = [tuple(plsc.sort_key_val(v, e, descending=True))
                     for v, e in chunks]
            while len(level) > 1:
                nxt = []
                for i in range(0, len(level) - 1, 2):
                    nxt.append(merge_top(level[i], level[i + 1])[0])
                if len(level) % 2:
                    nxt.append(level[-1])
                level = nxt
            cur_v, cur_e = level[0]

            sel_w = lax.shift_right_logical(cur_e, 6)
            sel_c = jnp.bitwise_and(cur_e, C - 1)
            g_pb = _vtake(pb, sel_w)
            g_spb = _vtake(s_pb, sel_w)
            g_spnb = _vtake(s_pnb, sel_w)
            g_tot = _vtake(tot, sel_w)
            g_last = _vtake(last, sel_w)
            g_len = _vtake(lens, sel_w)
            lp_sel = plsc.load_gather(lp_v, [_splat(t), sel_c])
            e_pnb = jnp.where(sel_c == g_last, g_pb, g_tot) + lp_sel
            is_stay = sel_c == 0
            new_pb = jnp.where(is_stay, g_spb, NEG).astype(jnp.float32)
            new_pnb = jnp.where(is_stay, g_spnb, e_pnb)
            new_last = jnp.where(is_stay, g_last, sel_c)
            new_len = jnp.where(is_stay, g_len, jnp.minimum(g_len + 1, T))
            bp_par[t] = sel_w
            bp_chr[t] = jnp.where(is_stay, -1, sel_c)
            return new_pb, new_pnb, new_last, new_len

        pb, pnb, last, lens = lax.fori_loop(
            0, my_len, step, (pb0, neg, last0, len0))

        score = _logaddexp(pb, pnb)
        sorted_sc, _ = plsc.sort_key_val(score, lanes, descending=True)
        mx = _vtake(sorted_sc, _splat(0))
        wstar = plsc.all_reduce_ffs(score == mx)
        pos0 = _vtake(lens, _splat(0) + wstar)

        for j in range(T // L):
            labels_v[pl.ds(j * L, L)] = jnp.zeros((L,), jnp.int32)

        def back(i, carry):
            wv, pos = carry
            t = my_len - 1 - i
            chr_ = plsc.load_gather(bp_chr, [_splat(t), wv])
            par = plsc.load_gather(bp_par, [_splat(t), wv])
            emit = chr_ >= 0
            npos = pos - jnp.where(emit, 1, 0)
            plsc.store_scatter(labels_v, [npos], chr_,
                               mask=jnp.logical_and(lanes == 0, emit))
            return par, npos

        lax.fori_loop(0, my_len, back, (_splat(0) + wstar, pos0))
        pltpu.sync_copy(labels_v, out_hbm.at[b])


@jax.jit
def kernel(data, data_length):
    lp = jnp.transpose(data, (1, 0, 2))
    mesh = plsc.VectorSubcoreMesh(core_axis_name="c", subcore_axis_name="s")
    f = pl.kernel(
        _body,
        out_type=jax.ShapeDtypeStruct((B, T), jnp.int32),
        mesh=mesh,
        compiler_params=pltpu.CompilerParams(needs_layout_passes=False),
        scratch_types=[
            pltpu.VMEM((T, C), jnp.float32),
            pltpu.VMEM((L,), jnp.int32),
            pltpu.VMEM((T, W), jnp.int32),
            pltpu.VMEM((T, W), jnp.int32),
            pltpu.VMEM((T,), jnp.int32),
        ],
    )
    return f(lp, data_length)
